# baseline (speedup 1.0000x reference)
import jax
import jax.numpy as jnp
from jax import lax
from jax.experimental import pallas as pl
from jax.experimental.pallas import tpu as pltpu
from jax.experimental.pallas import tpu_sc as plsc

HIDDEN = 64
VOCAB = 1000000
NUM_CORES = 2
NUM_SUBCORES = 16
NW = NUM_CORES * NUM_SUBCORES
B = 4096
T = 200

MB_COLS = 256
N_MB = VOCAB // MB_COLS
MB_PER_W = 122
TAIL_C0 = N_MB * MB_COLS

_MESH = plsc.VectorSubcoreMesh(core_axis_name="c", subcore_axis_name="s")


def _wid():
    return lax.axis_index("s") * NUM_CORES + lax.axis_index("c")


_IOTA = None


def _transpose_to_pairs(x_v, p_v, npairs):
    rows = [h0 + lax.iota(jnp.int32, 16) for h0 in (0, 16, 32, 48)]

    def grp(jj, carry):
        for dj in range(8):
            j = 8 * jj + dj
            for p in range(2):
                col = jnp.full((16,), 0, jnp.int32) + (2 * j + p)
                for hi, h0 in enumerate((0, 16, 32, 48)):
                    v = plsc.load_gather(x_v, [rows[hi], col])
                    p_v[j, pl.ds(64 * p + h0, 16)] = v
        return carry

    lax.fori_loop(0, npairs // 8, grp, 0)


def _k1_body(tab_t, tabP, x0, x1, p0, p1, xt, rs0, rs1, ws0, ws1):
    wid = _wid()
    base = wid * MB_PER_W
    xs = (x0, x1)
    ps = (p0, p1)
    rss = (rs0, rs1)
    wss = (ws0, ws1)

    def read(g, buf, rsem):
        c0 = (base + g) * MB_COLS
        pltpu.async_copy(tab_t.at[:, pl.ds(c0, MB_COLS)], buf, rsem)

    def write(g, buf, wsem):
        r0 = (base + g) * (MB_COLS // 2)
        pltpu.async_copy(buf, tabP.at[pl.ds(r0, MB_COLS // 2), :], wsem)

    read(0, x0, rs0)
    read(1, x1, rs1)

    def halfstep(g, b, wait_w):
        pltpu.make_async_copy(tab_t.at[:, pl.ds(0, MB_COLS)], xs[b],
                              rss[b]).wait()
        if wait_w:
            pltpu.make_async_copy(ps[b], tabP.at[pl.ds(0, MB_COLS // 2), :],
                                  wss[b]).wait()
        _transpose_to_pairs(xs[b], ps[b], 128)
        nxt = jnp.minimum(g + 2, MB_PER_W - 1)
        read(nxt, xs[b], rss[b])
        write(g, ps[b], wss[b])

    halfstep(0, 0, False)
    halfstep(1, 1, False)

    def pair(gg, carry):
        g0 = 2 * gg
        halfstep(g0, 0, True)
        halfstep(g0 + 1, 1, True)
        return carry

    lax.fori_loop(1, MB_PER_W // 2, pair, 0)

    pltpu.make_async_copy(tab_t.at[:, pl.ds(0, MB_COLS)], x0, rs0).wait()
    pltpu.make_async_copy(tab_t.at[:, pl.ds(0, MB_COLS)], x1, rs1).wait()
    pltpu.make_async_copy(p0, tabP.at[pl.ds(0, MB_COLS // 2), :], ws0).wait()
    pltpu.make_async_copy(p1, tabP.at[pl.ds(0, MB_COLS // 2), :], ws1).wait()

    @pl.when(wid < N_MB - NW * MB_PER_W)
    def _extra():
        mb = NW * MB_PER_W + wid
        pltpu.sync_copy(tab_t.at[:, pl.ds(mb * MB_COLS, MB_COLS)], x0)
        _transpose_to_pairs(x0, p0, 128)
        pltpu.sync_copy(p0, tabP.at[pl.ds(mb * (MB_COLS // 2), MB_COLS // 2), :])

    @pl.when(wid == NW - 1)
    def _tail():
        c0 = TAIL_C0 + lax.axis_index("c") * 0
        pltpu.sync_copy(tab_t.at[:, pl.ds(c0, 128)], xt)
        _transpose_to_pairs(xt, p0, 32)
        pltpu.sync_copy(p0.at[pl.ds(0, 32), :],
                        tabP.at[pl.ds(TAIL_C0 // 2, 32), :])


def _extract(g_v, o_v):
    rows = [16 * q + lax.iota(jnp.int32, 16) for q in range(8)]

    def stripe(i, carry):
        for r in range(8):
            col = jnp.full((16,), 0, jnp.int32) + (8 * i + r)
            for q in range(8):
                v = plsc.load_gather(g_v, [rows[q], col])
                o_v[i, r, pl.ds(16 * q, 16)] = v
        return carry

    lax.fori_loop(0, 8, stripe, 0)


def _k2_body(ids2d, tabv, out5, ids_v, g0, g1, o0, o1, gs0, gs1, ws0, ws1):
    wid = _wid()
    b0 = wid * 128
    pltpu.sync_copy(ids2d.at[:, pl.ds(b0, 128)], ids_v)
    gs = (g0, g1)
    os_ = (o0, o1)
    gss = (gs0, gs1)
    wss = (ws0, ws1)

    def gather(t, b):
        pltpu.async_copy(tabv.at[ids_v.at[t]], gs[b], gss[b])

    def write(t, b):
        pltpu.async_copy(os_[b], out5.at[t, :, wid, :, :], wss[b])

    gather(0, 0)
    gather(1, 1)

    def halfstep(t, b, wait_w):
        pltpu.make_async_copy(tabv.at[ids_v.at[0]], gs[b], gss[b]).wait()
        if wait_w:
            pltpu.make_async_copy(os_[b], out5.at[0, :, wid, :, :],
                                  wss[b]).wait()
        _extract(gs[b], os_[b])
        gather(jnp.minimum(t + 2, T - 1), b)
        write(t, b)

    halfstep(0, 0, False)
    halfstep(1, 1, False)

    def pair(tt, carry):
        t0 = 2 * tt
        halfstep(t0, 0, True)
        halfstep(t0 + 1, 1, True)
        return carry

    lax.fori_loop(1, T // 2, pair, 0)

    pltpu.make_async_copy(tabv.at[ids_v.at[0]], g0, gs0).wait()
    pltpu.make_async_copy(tabv.at[ids_v.at[0]], g1, gs1).wait()
    pltpu.make_async_copy(o0, out5.at[0, :, wid, :, :], ws0).wait()
    pltpu.make_async_copy(o1, out5.at[0, :, wid, :, :], ws1).wait()


@jax.jit
def _embed(ids, table):
    tab_t = table.T

    k1 = pl.kernel(
        _k1_body,
        mesh=_MESH,
        out_type=jax.ShapeDtypeStruct((VOCAB // 2, 128), jnp.float32),
        scratch_types=[
            pltpu.VMEM((HIDDEN, MB_COLS), jnp.float32),
            pltpu.VMEM((HIDDEN, MB_COLS), jnp.float32),
            pltpu.VMEM((128, 128), jnp.float32),
            pltpu.VMEM((128, 128), jnp.float32),
            pltpu.VMEM((HIDDEN, 128), jnp.float32),
            pltpu.SemaphoreType.DMA,
            pltpu.SemaphoreType.DMA,
            pltpu.SemaphoreType.DMA,
            pltpu.SemaphoreType.DMA,
        ],
        compiler_params=pltpu.CompilerParams(use_tc_tiling_on_sc=True,
                                             needs_layout_passes=False),
    )
    tabP = k1(tab_t)
    tabv = tabP.reshape(VOCAB, HIDDEN)

    k2 = pl.kernel(
        _k2_body,
        mesh=_MESH,
        out_type=jax.ShapeDtypeStruct((T, 8, 32, 8, 128), jnp.float32),
        scratch_types=[
            pltpu.VMEM((T, 128), jnp.int32),
            pltpu.VMEM((128, HIDDEN), jnp.float32),
            pltpu.VMEM((128, HIDDEN), jnp.float32),
            pltpu.VMEM((8, 8, 128), jnp.float32),
            pltpu.VMEM((8, 8, 128), jnp.float32),
            pltpu.SemaphoreType.DMA,
            pltpu.SemaphoreType.DMA,
            pltpu.SemaphoreType.DMA,
            pltpu.SemaphoreType.DMA,
        ],
        compiler_params=pltpu.CompilerParams(use_tc_tiling_on_sc=False,
                                             needs_layout_passes=False),
    )
    out5 = k2(ids.T, tabv)
    return jnp.transpose(out5, (2, 4, 0, 1, 3)).reshape(B, T, HIDDEN)


def kernel(input_ids, embed_tokens_weight):
    return _embed(input_ids.astype(jnp.int32), embed_tokens_weight)


# speedup vs baseline: 1.2405x; 1.2405x over previous
import jax
import jax.numpy as jnp
from jax import lax
from jax.experimental import pallas as pl
from jax.experimental.pallas import tpu as pltpu
from jax.experimental.pallas import tpu_sc as plsc

HIDDEN = 64
VOCAB = 1000000
NUM_CORES = 2
NUM_SUBCORES = 16
NW = NUM_CORES * NUM_SUBCORES
B = 4096
T = 200

MB_COLS = 256
N_MB = VOCAB // MB_COLS
MB_PER_W = 122
TAIL_C0 = N_MB * MB_COLS

_MESH = plsc.VectorSubcoreMesh(core_axis_name="c", subcore_axis_name="s")


def _wid():
    return lax.axis_index("s") * NUM_CORES + lax.axis_index("c")


_IOTA = None


def _transpose_to_pairs(x_v, p_v, npairs):
    ncols = 2 * npairs
    nq = ncols // 16
    iot = lax.iota(jnp.int32, 16)
    rows = [lax.shift_right_logical(16 * q + iot, 1) for q in range(nq)]
    cols = [lax.shift_left(lax.bitwise_and(16 * q + iot, 1), 6)
            for q in range(nq)]

    def hgrp(hh, carry):
        for dh in range(4):
            h = 4 * hh + dh
            hv = jnp.full((16,), 0, jnp.int32) + h
            for q in range(nq):
                v = x_v[h, pl.ds(16 * q, 16)]
                plsc.store_scatter(p_v, [rows[q], cols[q] + hv], v)
        return carry

    lax.fori_loop(0, HIDDEN // 4, hgrp, 0)


def _k1_body(tab_t, tabP, x0, x1, p0, p1, xt, rs0, rs1, ws0, ws1):
    wid = _wid()
    base = wid * MB_PER_W
    xs = (x0, x1)
    ps = (p0, p1)
    rss = (rs0, rs1)
    wss = (ws0, ws1)

    def read(g, buf, rsem):
        c0 = (base + g) * MB_COLS
        pltpu.async_copy(tab_t.at[:, pl.ds(c0, MB_COLS)], buf, rsem)

    def write(g, buf, wsem):
        r0 = (base + g) * (MB_COLS // 2)
        pltpu.async_copy(buf, tabP.at[pl.ds(r0, MB_COLS // 2), :], wsem)

    read(0, x0, rs0)
    read(1, x1, rs1)

    def halfstep(g, b, wait_w):
        pltpu.make_async_copy(tab_t.at[:, pl.ds(0, MB_COLS)], xs[b],
                              rss[b]).wait()
        if wait_w:
            pltpu.make_async_copy(ps[b], tabP.at[pl.ds(0, MB_COLS // 2), :],
                                  wss[b]).wait()
        _transpose_to_pairs(xs[b], ps[b], 128)
        nxt = jnp.minimum(g + 2, MB_PER_W - 1)
        read(nxt, xs[b], rss[b])
        write(g, ps[b], wss[b])

    halfstep(0, 0, False)
    halfstep(1, 1, False)

    def pair(gg, carry):
        g0 = 2 * gg
        halfstep(g0, 0, True)
        halfstep(g0 + 1, 1, True)
        return carry

    lax.fori_loop(1, MB_PER_W // 2, pair, 0)

    pltpu.make_async_copy(tab_t.at[:, pl.ds(0, MB_COLS)], x0, rs0).wait()
    pltpu.make_async_copy(tab_t.at[:, pl.ds(0, MB_COLS)], x1, rs1).wait()
    pltpu.make_async_copy(p0, tabP.at[pl.ds(0, MB_COLS // 2), :], ws0).wait()
    pltpu.make_async_copy(p1, tabP.at[pl.ds(0, MB_COLS // 2), :], ws1).wait()

    @pl.when(wid < N_MB - NW * MB_PER_W)
    def _extra():
        mb = NW * MB_PER_W + wid
        pltpu.sync_copy(tab_t.at[:, pl.ds(mb * MB_COLS, MB_COLS)], x0)
        _transpose_to_pairs(x0, p0, 128)
        pltpu.sync_copy(p0, tabP.at[pl.ds(mb * (MB_COLS // 2), MB_COLS // 2), :])

    @pl.when(wid == NW - 1)
    def _tail():
        c0 = TAIL_C0 + lax.axis_index("c") * 0
        pltpu.sync_copy(tab_t.at[:, pl.ds(c0, 128)], xt)
        _transpose_to_pairs(xt, p0, 32)
        pltpu.sync_copy(p0.at[pl.ds(0, 32), :],
                        tabP.at[pl.ds(TAIL_C0 // 2, 32), :])


def _extract(g_v, o_v):
    iot = lax.iota(jnp.int32, 16)
    rows = [lax.shift_right_logical(16 * q + iot, 3) for q in range(4)]
    cols = [lax.shift_left(lax.bitwise_and(16 * q + iot, 7), 7)
            for q in range(4)]

    def cgrp(cc, carry):
        for dc in range(8):
            c = 8 * cc + dc
            cv = jnp.full((16,), 0, jnp.int32) + c
            for q in range(4):
                v = g_v[c, pl.ds(16 * q, 16)]
                plsc.store_scatter(o_v, [rows[q], cols[q] + cv], v)
        return carry

    lax.fori_loop(0, 16, cgrp, 0)


def _k2_body(ids2d, tabv, out5, ids_v, g0, g1, o0, o1, gs0, gs1, ws0, ws1):
    wid = _wid()
    b0 = wid * 128
    pltpu.sync_copy(ids2d.at[:, pl.ds(b0, 128)], ids_v)
    gs = (g0, g1)
    os_ = (o0, o1)
    gss = (gs0, gs1)
    wss = (ws0, ws1)

    def gather(t, b):
        pltpu.async_copy(tabv.at[ids_v.at[t]], gs[b], gss[b])

    def write(t, b):
        pltpu.async_copy(os_[b], out5.at[t, :, wid, :], wss[b])

    gather(0, 0)
    gather(1, 1)

    def halfstep(t, b, wait_w):
        pltpu.make_async_copy(tabv.at[ids_v.at[0]], gs[b], gss[b]).wait()
        if wait_w:
            pltpu.make_async_copy(os_[b], out5.at[0, :, wid, :],
                                  wss[b]).wait()
        _extract(gs[b], os_[b])
        gather(jnp.minimum(t + 2, T - 1), b)
        write(t, b)

    halfstep(0, 0, False)
    halfstep(1, 1, False)

    def pair(tt, carry):
        t0 = 2 * tt
        halfstep(t0, 0, True)
        halfstep(t0 + 1, 1, True)
        return carry

    lax.fori_loop(1, T // 2, pair, 0)

    pltpu.make_async_copy(tabv.at[ids_v.at[0]], g0, gs0).wait()
    pltpu.make_async_copy(tabv.at[ids_v.at[0]], g1, gs1).wait()
    pltpu.make_async_copy(o0, out5.at[0, :, wid, :], ws0).wait()
    pltpu.make_async_copy(o1, out5.at[0, :, wid, :], ws1).wait()


@jax.jit
def _embed(ids, table):
    tab_t = table.T

    k1 = pl.kernel(
        _k1_body,
        mesh=_MESH,
        out_type=jax.ShapeDtypeStruct((VOCAB // 2, 128), jnp.float32),
        scratch_types=[
            pltpu.VMEM((HIDDEN, MB_COLS), jnp.float32),
            pltpu.VMEM((HIDDEN, MB_COLS), jnp.float32),
            pltpu.VMEM((128, 128), jnp.float32),
            pltpu.VMEM((128, 128), jnp.float32),
            pltpu.VMEM((HIDDEN, 128), jnp.float32),
            pltpu.SemaphoreType.DMA,
            pltpu.SemaphoreType.DMA,
            pltpu.SemaphoreType.DMA,
            pltpu.SemaphoreType.DMA,
        ],
        compiler_params=pltpu.CompilerParams(use_tc_tiling_on_sc=True,
                                             needs_layout_passes=False,
                                             disable_bounds_checks=True),
    )
    tabP = k1(tab_t)
    tabv = tabP.reshape(VOCAB, HIDDEN)

    k2 = pl.kernel(
        _k2_body,
        mesh=_MESH,
        out_type=jax.ShapeDtypeStruct((T, 8, 32, 1024), jnp.float32),
        scratch_types=[
            pltpu.VMEM((T, 128), jnp.int32),
            pltpu.VMEM((128, HIDDEN), jnp.float32),
            pltpu.VMEM((128, HIDDEN), jnp.float32),
            pltpu.VMEM((8, 1024), jnp.float32),
            pltpu.VMEM((8, 1024), jnp.float32),
            pltpu.SemaphoreType.DMA,
            pltpu.SemaphoreType.DMA,
            pltpu.SemaphoreType.DMA,
            pltpu.SemaphoreType.DMA,
        ],
        compiler_params=pltpu.CompilerParams(use_tc_tiling_on_sc=False,
                                             needs_layout_passes=False,
                                             disable_bounds_checks=True),
    )
    out5 = k2(ids.T, tabv)
    out6 = out5.reshape(T, 8, 32, 8, 128)
    return jnp.transpose(out6, (2, 4, 0, 1, 3)).reshape(B, T, HIDDEN)


def kernel(input_ids, embed_tokens_weight):
    return _embed(input_ids.astype(jnp.int32), embed_tokens_weight)


# speedup vs baseline: 1.7331x; 1.3971x over previous
import jax
import jax.numpy as jnp
from jax import lax
from jax.experimental import pallas as pl
from jax.experimental.pallas import tpu as pltpu
from jax.experimental.pallas import tpu_sc as plsc

HIDDEN = 64
VOCAB = 1000000
NUM_CORES = 2
NUM_SUBCORES = 16
NW = NUM_CORES * NUM_SUBCORES
B = 4096
T = 200

MB_COLS = 256
N_MB = VOCAB // MB_COLS
MB_PER_W = 122
TAIL_C0 = N_MB * MB_COLS

_MESH = plsc.VectorSubcoreMesh(core_axis_name="c", subcore_axis_name="s")


def _wid():
    return lax.axis_index("s") * NUM_CORES + lax.axis_index("c")


_IOTA = None


def _transpose_to_pairs(x_v, p_v, npairs):
    ncols = 2 * npairs
    nq = ncols // 16
    iot = lax.iota(jnp.int32, 16)
    rows = [lax.shift_right_logical(16 * q + iot, 1) for q in range(nq)]
    cols = [lax.shift_left(lax.bitwise_and(16 * q + iot, 1), 6)
            for q in range(nq)]

    @plsc.parallel_loop(0, HIDDEN, unroll=4)
    def _hstep(h):
        hv = jnp.full((16,), 0, jnp.int32) + h
        for q in range(nq):
            v = x_v[h, pl.ds(16 * q, 16)]
            plsc.store_scatter(p_v, [rows[q], cols[q] + hv], v)


def _k1_body(tab_t, tabP, x0, x1, p0, p1, xt, rs0, rs1, ws0, ws1):
    wid = _wid()
    base = wid * MB_PER_W
    xs = (x0, x1)
    ps = (p0, p1)
    rss = (rs0, rs1)
    wss = (ws0, ws1)

    def read(g, buf, rsem):
        c0 = (base + g) * MB_COLS
        pltpu.async_copy(tab_t.at[:, pl.ds(c0, MB_COLS)], buf, rsem)

    def write(g, buf, wsem):
        r0 = (base + g) * (MB_COLS // 2)
        pltpu.async_copy(buf, tabP.at[pl.ds(r0, MB_COLS // 2), :], wsem)

    read(0, x0, rs0)
    read(1, x1, rs1)

    def halfstep(g, b, wait_w):
        pltpu.make_async_copy(tab_t.at[:, pl.ds(0, MB_COLS)], xs[b],
                              rss[b]).wait()
        if wait_w:
            pltpu.make_async_copy(ps[b], tabP.at[pl.ds(0, MB_COLS // 2), :],
                                  wss[b]).wait()
        _transpose_to_pairs(xs[b], ps[b], 128)
        nxt = jnp.minimum(g + 2, MB_PER_W - 1)
        read(nxt, xs[b], rss[b])
        write(g, ps[b], wss[b])

    halfstep(0, 0, False)
    halfstep(1, 1, False)

    def pair(gg, carry):
        g0 = 2 * gg
        halfstep(g0, 0, True)
        halfstep(g0 + 1, 1, True)
        return carry

    lax.fori_loop(1, MB_PER_W // 2, pair, 0)

    pltpu.make_async_copy(tab_t.at[:, pl.ds(0, MB_COLS)], x0, rs0).wait()
    pltpu.make_async_copy(tab_t.at[:, pl.ds(0, MB_COLS)], x1, rs1).wait()
    pltpu.make_async_copy(p0, tabP.at[pl.ds(0, MB_COLS // 2), :], ws0).wait()
    pltpu.make_async_copy(p1, tabP.at[pl.ds(0, MB_COLS // 2), :], ws1).wait()

    @pl.when(wid < N_MB - NW * MB_PER_W)
    def _extra():
        mb = NW * MB_PER_W + wid
        pltpu.sync_copy(tab_t.at[:, pl.ds(mb * MB_COLS, MB_COLS)], x0)
        _transpose_to_pairs(x0, p0, 128)
        pltpu.sync_copy(p0, tabP.at[pl.ds(mb * (MB_COLS // 2), MB_COLS // 2), :])

    @pl.when(wid == NW - 1)
    def _tail():
        c0 = TAIL_C0 + lax.axis_index("c") * 0
        pltpu.sync_copy(tab_t.at[:, pl.ds(c0, 128)], xt)
        _transpose_to_pairs(xt, p0, 32)
        pltpu.sync_copy(p0.at[pl.ds(0, 32), :],
                        tabP.at[pl.ds(TAIL_C0 // 2, 32), :])


def _extract(g_v, o_v):
    iot = lax.iota(jnp.int32, 16)
    rows = [lax.shift_right_logical(16 * q + iot, 3) for q in range(4)]
    cols = [lax.shift_left(lax.bitwise_and(16 * q + iot, 7), 7)
            for q in range(4)]

    @plsc.parallel_loop(0, 128, unroll=8)
    def _cstep(c):
        cv = jnp.full((16,), 0, jnp.int32) + c
        for q in range(4):
            v = g_v[c, pl.ds(16 * q, 16)]
            plsc.store_scatter(o_v, [rows[q], cols[q] + cv], v)


def _k2_body(ids2d, tabv, out5, ids_v, g0, g1, o0, o1, gs0, gs1, ws0, ws1):
    wid = _wid()
    b0 = wid * 128
    pltpu.sync_copy(ids2d.at[:, pl.ds(b0, 128)], ids_v)
    gs = (g0, g1)
    os_ = (o0, o1)
    gss = (gs0, gs1)
    wss = (ws0, ws1)

    def gather(t, b):
        pltpu.async_copy(tabv.at[ids_v.at[t]], gs[b], gss[b])

    def write(t, b):
        pltpu.async_copy(os_[b], out5.at[t, :, wid, :], wss[b])

    gather(0, 0)
    gather(1, 1)

    def halfstep(t, b, wait_w):
        pltpu.make_async_copy(tabv.at[ids_v.at[0]], gs[b], gss[b]).wait()
        if wait_w:
            pltpu.make_async_copy(os_[b], out5.at[0, :, wid, :],
                                  wss[b]).wait()
        _extract(gs[b], os_[b])
        gather(jnp.minimum(t + 2, T - 1), b)
        write(t, b)

    halfstep(0, 0, False)
    halfstep(1, 1, False)

    def pair(tt, carry):
        t0 = 2 * tt
        halfstep(t0, 0, True)
        halfstep(t0 + 1, 1, True)
        return carry

    lax.fori_loop(1, T // 2, pair, 0)

    pltpu.make_async_copy(tabv.at[ids_v.at[0]], g0, gs0).wait()
    pltpu.make_async_copy(tabv.at[ids_v.at[0]], g1, gs1).wait()
    pltpu.make_async_copy(o0, out5.at[0, :, wid, :], ws0).wait()
    pltpu.make_async_copy(o1, out5.at[0, :, wid, :], ws1).wait()


@jax.jit
def _embed(ids, table):
    tab_t = table.T

    k1 = pl.kernel(
        _k1_body,
        mesh=_MESH,
        out_type=jax.ShapeDtypeStruct((VOCAB // 2, 128), jnp.float32),
        scratch_types=[
            pltpu.VMEM((HIDDEN, MB_COLS), jnp.float32),
            pltpu.VMEM((HIDDEN, MB_COLS), jnp.float32),
            pltpu.VMEM((128, 128), jnp.float32),
            pltpu.VMEM((128, 128), jnp.float32),
            pltpu.VMEM((HIDDEN, 128), jnp.float32),
            pltpu.SemaphoreType.DMA,
            pltpu.SemaphoreType.DMA,
            pltpu.SemaphoreType.DMA,
            pltpu.SemaphoreType.DMA,
        ],
        compiler_params=pltpu.CompilerParams(use_tc_tiling_on_sc=True,
                                             needs_layout_passes=False,
                                             disable_bounds_checks=True),
    )
    tabP = k1(tab_t)
    tabv = tabP.reshape(VOCAB, HIDDEN)

    k2 = pl.kernel(
        _k2_body,
        mesh=_MESH,
        out_type=jax.ShapeDtypeStruct((T, 8, 32, 1024), jnp.float32),
        scratch_types=[
            pltpu.VMEM((T, 128), jnp.int32),
            pltpu.VMEM((128, HIDDEN), jnp.float32),
            pltpu.VMEM((128, HIDDEN), jnp.float32),
            pltpu.VMEM((8, 1024), jnp.float32),
            pltpu.VMEM((8, 1024), jnp.float32),
            pltpu.SemaphoreType.DMA,
            pltpu.SemaphoreType.DMA,
            pltpu.SemaphoreType.DMA,
            pltpu.SemaphoreType.DMA,
        ],
        compiler_params=pltpu.CompilerParams(use_tc_tiling_on_sc=False,
                                             needs_layout_passes=False,
                                             disable_bounds_checks=True),
    )
    out5 = k2(ids.T, tabv)
    out6 = out5.reshape(T, 8, 32, 8, 128)
    return jnp.transpose(out6, (2, 4, 0, 1, 3)).reshape(B, T, HIDDEN)


def kernel(input_ids, embed_tokens_weight):
    return _embed(input_ids.astype(jnp.int32), embed_tokens_weight)


# speedup vs baseline: 1.7333x; 1.0001x over previous
import jax
import jax.numpy as jnp
from jax import lax
from jax.experimental import pallas as pl
from jax.experimental.pallas import tpu as pltpu
from jax.experimental.pallas import tpu_sc as plsc

HIDDEN = 64
VOCAB = 1000000
NUM_CORES = 2
NUM_SUBCORES = 16
NW = NUM_CORES * NUM_SUBCORES
B = 4096
T = 200

MB_COLS = 256
N_MB = VOCAB // MB_COLS
MB_PER_W = 123
TAIL_C0 = N_MB * MB_COLS

_MESH = plsc.VectorSubcoreMesh(core_axis_name="c", subcore_axis_name="s")


def _wid():
    return lax.axis_index("s") * NUM_CORES + lax.axis_index("c")


_IOTA = None


def _transpose_to_pairs(x_v, p_v, npairs):
    ncols = 2 * npairs
    nq = ncols // 16
    iot = lax.iota(jnp.int32, 16)
    rows = [lax.shift_right_logical(16 * q + iot, 1) for q in range(nq)]
    cols = [lax.shift_left(lax.bitwise_and(16 * q + iot, 1), 6)
            for q in range(nq)]

    @plsc.parallel_loop(0, HIDDEN, unroll=4)
    def _hstep(h):
        hv = jnp.full((16,), 0, jnp.int32) + h
        for q in range(nq):
            v = x_v[h, pl.ds(16 * q, 16)]
            plsc.store_scatter(p_v, [rows[q], cols[q] + hv], v)


def _k1_body(tab_t, tabP, x0, x1, x2, p0, p1, p2, xt,
             rs0, rs1, rs2, ws0, ws1, ws2):
    wid = _wid()
    base = wid * MB_PER_W
    xs = (x0, x1, x2)
    ps = (p0, p1, p2)
    rss = (rs0, rs1, rs2)
    wss = (ws0, ws1, ws2)

    def mb(g):
        return jnp.minimum(base + g, N_MB - 1)

    def read(g, b):
        pltpu.async_copy(tab_t.at[:, pl.ds(mb(g) * MB_COLS, MB_COLS)],
                         xs[b], rss[b])

    def write(g, b):
        pltpu.async_copy(ps[b], tabP.at[pl.ds(mb(g) * (MB_COLS // 2),
                                              MB_COLS // 2), :], wss[b])

    for b in range(3):
        read(b, b)

    def step(g, b, wait_w):
        pltpu.make_async_copy(tab_t.at[:, pl.ds(0, MB_COLS)], xs[b],
                              rss[b]).wait()
        if wait_w:
            pltpu.make_async_copy(ps[b], tabP.at[pl.ds(0, MB_COLS // 2), :],
                                  wss[b]).wait()
        _transpose_to_pairs(xs[b], ps[b], 128)
        read(g + 3, b)
        write(g, b)

    step(0, 0, False)
    step(1, 1, False)
    step(2, 2, False)

    def tri(tt, carry):
        g = 3 * tt
        step(g, 0, True)
        step(g + 1, 1, True)
        step(g + 2, 2, True)
        return carry

    lax.fori_loop(1, MB_PER_W // 3, tri, 0)

    for b in range(3):
        pltpu.make_async_copy(tab_t.at[:, pl.ds(0, MB_COLS)], xs[b],
                              rss[b]).wait()
        pltpu.make_async_copy(ps[b], tabP.at[pl.ds(0, MB_COLS // 2), :],
                              wss[b]).wait()

    @pl.when(wid == NW - 1)
    def _tail():
        c0 = TAIL_C0 + lax.axis_index("c") * 0
        pltpu.sync_copy(tab_t.at[:, pl.ds(c0, 128)], xt)
        _transpose_to_pairs(xt, p0, 32)
        pltpu.sync_copy(p0.at[pl.ds(0, 32), :],
                        tabP.at[pl.ds(TAIL_C0 // 2, 32), :])


def _extract(g_v, o_v):
    iot = lax.iota(jnp.int32, 16)
    rows = [lax.shift_right_logical(16 * q + iot, 3) for q in range(4)]
    cols = [lax.shift_left(lax.bitwise_and(16 * q + iot, 7), 7)
            for q in range(4)]

    @plsc.parallel_loop(0, 128, unroll=8)
    def _cstep(c):
        cv = jnp.full((16,), 0, jnp.int32) + c
        for q in range(4):
            v = g_v[c, pl.ds(16 * q, 16)]
            plsc.store_scatter(o_v, [rows[q], cols[q] + cv], v)


def _k2_body(ids2d, tabv, out5, ids_v,
             g0, g1, g2, g3, o0, o1, o2, o3,
             gs0, gs1, gs2, gs3, ws0, ws1, ws2, ws3):
    wid = _wid()
    b0 = wid * 128
    pltpu.sync_copy(ids2d.at[:, pl.ds(b0, 128)], ids_v)
    gs = (g0, g1, g2, g3)
    os_ = (o0, o1, o2, o3)
    gss = (gs0, gs1, gs2, gs3)
    wss = (ws0, ws1, ws2, ws3)

    def gather(t, b):
        pltpu.async_copy(tabv.at[ids_v.at[jnp.minimum(t, T - 1)]],
                         gs[b], gss[b])

    def write(t, b):
        pltpu.async_copy(os_[b], out5.at[t, :, wid, :], wss[b])

    for b in range(4):
        gather(b, b)

    def step(t, b, wait_w):
        pltpu.make_async_copy(tabv.at[ids_v.at[0]], gs[b], gss[b]).wait()
        if wait_w:
            pltpu.make_async_copy(os_[b], out5.at[0, :, wid, :],
                                  wss[b]).wait()
        _extract(gs[b], os_[b])
        gather(t + 4, b)
        write(t, b)

    for b in range(4):
        step(b, b, False)

    def quad(tt, carry):
        t0 = 4 * tt
        for b in range(4):
            step(t0 + b, b, True)
        return carry

    lax.fori_loop(1, T // 4, quad, 0)

    for b in range(4):
        pltpu.make_async_copy(tabv.at[ids_v.at[0]], gs[b], gss[b]).wait()
        pltpu.make_async_copy(os_[b], out5.at[0, :, wid, :], wss[b]).wait()


@jax.jit
def _embed(ids, table):
    tab_t = table.T

    k1 = pl.kernel(
        _k1_body,
        mesh=_MESH,
        out_type=jax.ShapeDtypeStruct((VOCAB // 2, 128), jnp.float32),
        scratch_types=[
            pltpu.VMEM((HIDDEN, MB_COLS), jnp.float32),
            pltpu.VMEM((HIDDEN, MB_COLS), jnp.float32),
            pltpu.VMEM((HIDDEN, MB_COLS), jnp.float32),
            pltpu.VMEM((128, 128), jnp.float32),
            pltpu.VMEM((128, 128), jnp.float32),
            pltpu.VMEM((128, 128), jnp.float32),
            pltpu.VMEM((HIDDEN, 128), jnp.float32),
        ] + [pltpu.SemaphoreType.DMA] * 6,
        compiler_params=pltpu.CompilerParams(use_tc_tiling_on_sc=True,
                                             needs_layout_passes=False,
                                             disable_bounds_checks=True),
    )
    tabP = k1(tab_t)
    tabv = tabP.reshape(VOCAB, HIDDEN)

    k2 = pl.kernel(
        _k2_body,
        mesh=_MESH,
        out_type=jax.ShapeDtypeStruct((T, 8, 32, 1024), jnp.float32),
        scratch_types=[
            pltpu.VMEM((T, 128), jnp.int32),
        ] + [pltpu.VMEM((128, HIDDEN), jnp.float32)] * 4
          + [pltpu.VMEM((8, 1024), jnp.float32)] * 4
          + [pltpu.SemaphoreType.DMA] * 8,
        compiler_params=pltpu.CompilerParams(use_tc_tiling_on_sc=False,
                                             needs_layout_passes=False,
                                             disable_bounds_checks=True),
    )
    out5 = k2(ids.T, tabv)
    out6 = out5.reshape(T, 8, 32, 8, 128)
    return jnp.transpose(out6, (2, 4, 0, 1, 3)).reshape(B, T, HIDDEN)


def kernel(input_ids, embed_tokens_weight):
    return _embed(input_ids.astype(jnp.int32), embed_tokens_weight)
